# baseline (speedup 1.0000x reference)
import functools

import jax
import jax.numpy as jnp
from jax import lax
from jax.experimental import pallas as pl
from jax.experimental.pallas import tpu as pltpu
from jax.experimental.pallas import tpu_sc as plsc

N = 10000
E = 320000
D_IN = 128
H = 256
C1 = 8
C2 = 32
C3 = 128
G = 64

NC = 2
NS = 16

N_Z = 10240
ROWS_T = N_Z // NS
CHUNK = 128
E_PAD = 327680
EROWS = E_PAD // CHUNK
MROWS = EROWS // NS
DROWS = EROWS // (NS * NC)
GR = 32
GROUPS = MROWS // GR

BN = 512
NB = N_Z // BN


def _sc_mesh():
  return plsc.VectorSubcoreMesh(core_axis_name="c", subcore_axis_name="s")


def _sc_degree(dst2d, zeros128, ones128):
  @functools.partial(
      pl.kernel,
      out_type=jax.ShapeDtypeStruct((NC, N_Z, 128), jnp.float32),
      mesh=_sc_mesh(),
      scratch_types=[
          pltpu.VMEM_SHARED((N_Z, 128), jnp.float32),
          pltpu.VMEM((DROWS, CHUNK), jnp.int32),
          pltpu.VMEM((CHUNK, 128), jnp.float32),
          pltpu.SemaphoreType.DMA,
      ],
  )
  def k(dst_hbm, zeros_hbm, ones_hbm, out_hbm, deg_sh, dst_v, ones_v, ssem):
    c = lax.axis_index("c")
    s = lax.axis_index("s")
    wid = s * NC + c
    pltpu.sync_copy(zeros_hbm.at[pl.ds(s * ROWS_T, ROWS_T), :],
                    deg_sh.at[pl.ds(s * ROWS_T, ROWS_T), :])
    pltpu.sync_copy(ones_hbm, ones_v)
    pltpu.sync_copy(dst_hbm.at[pl.ds(wid * DROWS, DROWS), :], dst_v)
    plsc.subcore_barrier()

    for j in range(4):
      pltpu.async_copy(ones_v, deg_sh.at[dst_v.at[j]], ssem, add=True)

    def body(j, carry):
      pltpu.make_async_copy(ones_v, deg_sh.at[dst_v.at[0]], ssem).wait()
      pltpu.async_copy(ones_v, deg_sh.at[dst_v.at[j]], ssem, add=True)
      return carry

    lax.fori_loop(4, DROWS, body, 0)
    for _ in range(4):
      pltpu.make_async_copy(ones_v, deg_sh.at[dst_v.at[0]], ssem).wait()
    plsc.subcore_barrier()
    pltpu.sync_copy(deg_sh.at[pl.ds(s * ROWS_T, ROWS_T), :],
                    out_hbm.at[c, pl.ds(s * ROWS_T, ROWS_T), :])

  return k(dst2d, zeros128, ones128)


def _sc_edge_sum(y, src2d, dst2d, zeros128):
  @functools.partial(
      pl.kernel,
      out_type=jax.ShapeDtypeStruct((NC, N_Z, 128), jnp.float32),
      mesh=_sc_mesh(),
      scratch_types=[
          pltpu.VMEM_SHARED((N_Z, 128), jnp.float32),
          pltpu.VMEM((GR, CHUNK), jnp.int32),
          pltpu.VMEM((GR, CHUNK), jnp.int32),
          pltpu.VMEM((CHUNK, 128), jnp.float32),
          pltpu.VMEM((CHUNK, 128), jnp.float32),
          pltpu.SemaphoreType.DMA,
          pltpu.SemaphoreType.DMA,
          pltpu.SemaphoreType.DMA,
          pltpu.SemaphoreType.DMA,
      ],
  )
  def k(y_hbm, src_hbm, dst_hbm, zeros_hbm, out_hbm,
        z_sh, src_v, dst_v, buf0, buf1, gsem0, gsem1, ssem0, ssem1):
    c = lax.axis_index("c")
    s = lax.axis_index("s")
    pltpu.sync_copy(zeros_hbm.at[pl.ds(s * ROWS_T, ROWS_T), :],
                    z_sh.at[pl.ds(s * ROWS_T, ROWS_T), :])
    plsc.subcore_barrier()
    y2 = y_hbm.at[c]

    def group(g, carry):
      base = s * MROWS + g * GR
      pltpu.sync_copy(src_hbm.at[pl.ds(base, GR), :], src_v)
      pltpu.sync_copy(dst_hbm.at[pl.ds(base, GR), :], dst_v)
      def start_gather(r, buf, gsem):
        hh = CHUNK // 2
        pltpu.async_copy(y2.at[src_v.at[r, pl.ds(0, hh)]],
                         buf.at[pl.ds(0, hh), :], gsem)
        pltpu.async_copy(y2.at[src_v.at[r, pl.ds(hh, hh)]],
                         buf.at[pl.ds(hh, hh), :], gsem)

      def wait_gather(r, buf, gsem):
        pltpu.make_async_copy(y2.at[src_v.at[r]], buf, gsem).wait()

      start_gather(0, buf0, gsem0)
      start_gather(1, buf1, gsem1)

      def pair(jj, carry2):
        r0 = jj * 2
        r1 = r0 + 1
        wait_gather(r0, buf0, gsem0)
        pltpu.async_copy(buf0, z_sh.at[dst_v.at[r0]], ssem0, add=True)
        wait_gather(r1, buf1, gsem1)
        pltpu.async_copy(buf1, z_sh.at[dst_v.at[r1]], ssem1, add=True)
        pltpu.make_async_copy(buf0, z_sh.at[dst_v.at[r0]], ssem0).wait()
        start_gather(r0 + 2, buf0, gsem0)
        pltpu.make_async_copy(buf1, z_sh.at[dst_v.at[r1]], ssem1).wait()
        start_gather(r1 + 2, buf1, gsem1)
        return carry2

      lax.fori_loop(0, GR // 2 - 1, pair, carry)
      r0 = GR - 2
      r1 = GR - 1
      wait_gather(r0, buf0, gsem0)
      pltpu.sync_copy(buf0, z_sh.at[dst_v.at[r0]], add=True)
      wait_gather(r1, buf1, gsem1)
      pltpu.sync_copy(buf1, z_sh.at[dst_v.at[r1]], add=True)
      return carry

    lax.fori_loop(0, GROUPS, group, 0)
    plsc.subcore_barrier()
    pltpu.sync_copy(z_sh.at[pl.ds(s * ROWS_T, ROWS_T), :],
                    out_hbm.at[c, pl.ds(s * ROWS_T, ROWS_T), :])

  return k(y, src2d, dst2d, zeros128)


def _dis_from(d_ref):
  deg = d_ref[0, :, 0:1] + d_ref[1, :, 0:1] + 1.0
  return lax.rsqrt(deg)


def _tc_matmul1(x_pad, W1):
  def body(x_ref, w_ref, xw_ref):
    xw = jnp.dot(x_ref[...], w_ref[...], preferred_element_type=jnp.float32)
    xw_ref[0] = xw[:, :128]
    xw_ref[1] = xw[:, 128:]

  return pl.pallas_call(
      body,
      grid=(NB,),
      in_specs=[
          pl.BlockSpec((BN, D_IN), lambda i: (i, 0)),
          pl.BlockSpec((D_IN, H), lambda i: (0, 0)),
      ],
      out_specs=pl.BlockSpec((NC, BN, 128), lambda i: (0, i, 0)),
      out_shape=jax.ShapeDtypeStruct((NC, N_Z, 128), jnp.float32),
  )(x_pad, W1)


def _tc_scale_y(xw, degs):
  def body(xw_ref, d_ref, y_ref):
    dis = _dis_from(d_ref)
    y_ref[0] = dis * xw_ref[0]
    y_ref[1] = dis * xw_ref[1]

  return pl.pallas_call(
      body,
      grid=(NB,),
      in_specs=[
          pl.BlockSpec((NC, BN, 128), lambda i: (0, i, 0)),
          pl.BlockSpec((NC, BN, 128), lambda i: (0, i, 0)),
      ],
      out_specs=pl.BlockSpec((NC, BN, 128), lambda i: (0, i, 0)),
      out_shape=jax.ShapeDtypeStruct((NC, N_Z, 128), jnp.float32),
  )(xw, degs)


def _tc_boundary(z, xw, degs, b, Wn, relu):
  def body(z_ref, xw_ref, d_ref, b_ref, w_ref, y_ref, xwn_ref):
    dis = _dis_from(d_ref)
    z_full = jnp.concatenate([z_ref[0], z_ref[1]], axis=1)
    xw_full = jnp.concatenate([xw_ref[0], xw_ref[1]], axis=1)
    h = dis * z_full + (dis * dis) * xw_full + b_ref[...]
    if relu:
      h = jnp.maximum(h, 0.0)
    xwn = jnp.dot(h, w_ref[...], preferred_element_type=jnp.float32)
    y = dis * xwn
    xwn_ref[0] = xwn[:, :128]
    xwn_ref[1] = xwn[:, 128:]
    y_ref[0] = y[:, :128]
    y_ref[1] = y[:, 128:]

  return pl.pallas_call(
      body,
      grid=(NB,),
      in_specs=[
          pl.BlockSpec((NC, BN, 128), lambda i: (0, i, 0)),
          pl.BlockSpec((NC, BN, 128), lambda i: (0, i, 0)),
          pl.BlockSpec((NC, BN, 128), lambda i: (0, i, 0)),
          pl.BlockSpec((1, H), lambda i: (0, 0)),
          pl.BlockSpec((H, H), lambda i: (0, 0)),
      ],
      out_specs=[
          pl.BlockSpec((NC, BN, 128), lambda i: (0, i, 0)),
          pl.BlockSpec((NC, BN, 128), lambda i: (0, i, 0)),
      ],
      out_shape=[
          jax.ShapeDtypeStruct((NC, N_Z, 128), jnp.float32),
          jax.ShapeDtypeStruct((NC, N_Z, 128), jnp.float32),
      ],
  )(z, xw, degs, b, Wn)


def _tc_head(z, xw, degs, b3, batch8,
             Wl1, bl1, Wl2a, Wl2b, bl2, Wl3a, Wl3b, bl3):
  def body(z_ref, xw_ref, d_ref, b3_ref, bt_ref,
           wl1_ref, bl1_ref, wl2a_ref, wl2b_ref, bl2_ref,
           wl3a_ref, wl3b_ref, bl3_ref,
           l1_ref, l2_ref, l3_ref, acc_sum, acc_cnt):
    i = pl.program_id(0)
    dis = _dis_from(d_ref)
    z_full = jnp.concatenate([z_ref[0], z_ref[1]], axis=1)
    xw_full = jnp.concatenate([xw_ref[0], xw_ref[1]], axis=1)
    h = dis * z_full + (dis * dis) * xw_full + b3_ref[...]

    gids = lax.broadcasted_iota(jnp.int32, (G, 1), 0)
    onehot_t = (bt_ref[0:1, :] == gids).astype(jnp.float32)

    @pl.when(i == 0)
    def _():
      acc_sum[...] = jnp.zeros_like(acc_sum)
      acc_cnt[...] = jnp.zeros_like(acc_cnt)

    acc_sum[...] += jnp.dot(onehot_t, h, preferred_element_type=jnp.float32)
    cnt = jnp.sum(onehot_t, axis=1, keepdims=True)
    acc_cnt[...] += jnp.broadcast_to(cnt, (G, 128))

    @pl.when(i == NB - 1)
    def _():
      pooled = acc_sum[...] / jnp.maximum(acc_cnt[:, 0:1], 1.0)

      def softmax(v):
        m = jnp.max(v, axis=1, keepdims=True)
        e = jnp.exp(v - m)
        return e / jnp.sum(e, axis=1, keepdims=True)

      l1 = jnp.dot(pooled, wl1_ref[...],
                   preferred_element_type=jnp.float32) + bl1_ref[...]
      p1 = softmax(l1)
      l2 = (jnp.dot(pooled, wl2a_ref[...], preferred_element_type=jnp.float32)
            + jnp.dot(p1, wl2b_ref[...], preferred_element_type=jnp.float32)
            + bl2_ref[...])
      p2 = softmax(l2)
      l3 = (jnp.dot(pooled, wl3a_ref[...], preferred_element_type=jnp.float32)
            + jnp.dot(p2, wl3b_ref[...], preferred_element_type=jnp.float32)
            + bl3_ref[...])
      l1_ref[...] = l1
      l2_ref[...] = l2
      l3_ref[...] = l3

  full = lambda shape: pl.BlockSpec(shape, lambda i: tuple(0 for _ in shape))
  return pl.pallas_call(
      body,
      grid=(NB,),
      in_specs=[
          pl.BlockSpec((NC, BN, 128), lambda i: (0, i, 0)),
          pl.BlockSpec((NC, BN, 128), lambda i: (0, i, 0)),
          pl.BlockSpec((NC, BN, 128), lambda i: (0, i, 0)),
          full((1, H)),
          pl.BlockSpec((8, BN), lambda i: (0, i)),
          full((H, C1)),
          full((1, C1)),
          full((H, C2)),
          full((C1, C2)),
          full((1, C2)),
          full((H, C3)),
          full((C2, C3)),
          full((1, C3)),
      ],
      out_specs=[
          full((G, C1)),
          full((G, C2)),
          full((G, C3)),
      ],
      out_shape=[
          jax.ShapeDtypeStruct((G, C1), jnp.float32),
          jax.ShapeDtypeStruct((G, C2), jnp.float32),
          jax.ShapeDtypeStruct((G, C3), jnp.float32),
      ],
      scratch_shapes=[
          pltpu.VMEM((G, H), jnp.float32),
          pltpu.VMEM((G, 128), jnp.float32),
      ],
  )(z, xw, degs, b3, batch8,
    Wl1, bl1, Wl2a, Wl2b, bl2, Wl3a, Wl3b, bl3)


def kernel(x, edge_index, batch,
           W1, b1, W2, b2, W3, b3, Wl1, bl1, Wl2, bl2, Wl3, bl3):
  src = edge_index[0]
  dst = edge_index[1]
  npad = E_PAD - E
  ar = jnp.arange(npad, dtype=jnp.int32)
  pad_src = (ar * 13) % N
  pad_dst = N + (ar % (N_Z - N))
  src2d = jnp.concatenate([src, pad_src]).reshape(EROWS, CHUNK)
  dst2d = jnp.concatenate([dst, pad_dst]).reshape(EROWS, CHUNK)

  zeros128 = jnp.zeros((N_Z, 128), jnp.float32)
  ones128 = jnp.ones((CHUNK, 128), jnp.float32)
  x_pad = jnp.pad(x, ((0, N_Z - N), (0, 0)))
  batch_pad = jnp.pad(batch, (0, N_Z - N), constant_values=G)
  batch8 = jnp.broadcast_to(batch_pad[None, :], (8, N_Z))

  degs = _sc_degree(dst2d, zeros128, ones128)
  xw = _tc_matmul1(x_pad, W1)
  y = _tc_scale_y(xw, degs)
  z = _sc_edge_sum(y, src2d, dst2d, zeros128)
  y, xw = _tc_boundary(z, xw, degs, b1.reshape(1, H), W2, relu=True)
  z = _sc_edge_sum(y, src2d, dst2d, zeros128)
  y, xw = _tc_boundary(z, xw, degs, b2.reshape(1, H), W3, relu=True)
  z = _sc_edge_sum(y, src2d, dst2d, zeros128)
  l1, l2, l3 = _tc_head(
      z, xw, degs, b3.reshape(1, H), batch8,
      Wl1, bl1.reshape(1, C1), Wl2[:H], Wl2[H:], bl2.reshape(1, C2),
      Wl3[:H], Wl3[H:], bl3.reshape(1, C3))
  return (l1, l2, l3)


# speedup vs baseline: 1.0494x; 1.0494x over previous
import functools

import jax
import jax.numpy as jnp
from jax import lax
from jax.experimental import pallas as pl
from jax.experimental.pallas import tpu as pltpu
from jax.experimental.pallas import tpu_sc as plsc

N = 10000
E = 320000
D_IN = 128
H = 256
C1 = 8
C2 = 32
C3 = 128
G = 64

NC = 2
NS = 16

N_Z = 10240
ROWS_T = N_Z // NS
CHUNK = 128
E_PAD = 327680
EROWS = E_PAD // CHUNK
MROWS = EROWS // NS
DROWS = EROWS // (NS * NC)
GR = 32
GROUPS = MROWS // GR

BN = 512
NB = N_Z // BN


def _sc_mesh():
  return plsc.VectorSubcoreMesh(core_axis_name="c", subcore_axis_name="s")


def _sc_degree(dst2d):
  @functools.partial(
      pl.kernel,
      out_type=jax.ShapeDtypeStruct((NC * N_Z,), jnp.float32),
      mesh=_sc_mesh(),
      scratch_types=[
          pltpu.VMEM_SHARED((N_Z,), jnp.float32),
          pltpu.VMEM((DROWS, CHUNK), jnp.int32),
          pltpu.VMEM((CHUNK,), jnp.float32),
          pltpu.VMEM((ROWS_T,), jnp.float32),
          pltpu.SemaphoreType.DMA,
      ],
  )
  def k(dst_hbm, out_hbm, deg_sh, dst_v, ones_v, st640, ssem):
    c = lax.axis_index("c")
    s = lax.axis_index("s")
    wid = s * NC + c
    one16 = jnp.full((16,), 1.0, jnp.float32)
    zero16 = jnp.zeros((16,), jnp.float32)
    for i in range(CHUNK // 16):
      ones_v[pl.ds(i * 16, 16)] = one16
    for i in range(ROWS_T // 16):
      st640[pl.ds(i * 16, 16)] = zero16
    pltpu.sync_copy(st640, deg_sh.at[pl.ds(s * ROWS_T, ROWS_T)])
    pltpu.sync_copy(dst_hbm.at[pl.ds(wid * DROWS, DROWS), :], dst_v)
    plsc.subcore_barrier()

    for j in range(4):
      pltpu.async_copy(ones_v, deg_sh.at[dst_v.at[j]], ssem, add=True)

    def body(j, carry):
      pltpu.make_async_copy(ones_v, deg_sh.at[dst_v.at[0]], ssem).wait()
      pltpu.async_copy(ones_v, deg_sh.at[dst_v.at[j]], ssem, add=True)
      return carry

    lax.fori_loop(4, DROWS, body, 0)
    for _ in range(4):
      pltpu.make_async_copy(ones_v, deg_sh.at[dst_v.at[0]], ssem).wait()
    plsc.subcore_barrier()

    pltpu.sync_copy(deg_sh.at[pl.ds(s * ROWS_T, ROWS_T)],
                    out_hbm.at[pl.ds(c * N_Z + s * ROWS_T, ROWS_T)])

  return k(dst2d)


def _sc_edge_sum(y, src2d, dst2d, zeros128):
  @functools.partial(
      pl.kernel,
      out_type=jax.ShapeDtypeStruct((NC, N_Z, 128), jnp.float32),
      mesh=_sc_mesh(),
      scratch_types=[
          pltpu.VMEM_SHARED((N_Z, 128), jnp.float32),
          pltpu.VMEM((GR, CHUNK), jnp.int32),
          pltpu.VMEM((GR, CHUNK), jnp.int32),
          pltpu.VMEM((CHUNK, 128), jnp.float32),
          pltpu.VMEM((CHUNK, 128), jnp.float32),
          pltpu.SemaphoreType.DMA,
          pltpu.SemaphoreType.DMA,
          pltpu.SemaphoreType.DMA,
          pltpu.SemaphoreType.DMA,
      ],
  )
  def k(y_hbm, src_hbm, dst_hbm, zeros_hbm, out_hbm,
        z_sh, src_v, dst_v, buf0, buf1, gsem0, gsem1, ssem0, ssem1):
    c = lax.axis_index("c")
    s = lax.axis_index("s")
    pltpu.sync_copy(zeros_hbm.at[pl.ds(s * ROWS_T, ROWS_T), :],
                    z_sh.at[pl.ds(s * ROWS_T, ROWS_T), :])
    plsc.subcore_barrier()
    y2 = y_hbm.at[c]

    def group(g, carry):
      base = s * MROWS + g * GR
      pltpu.sync_copy(src_hbm.at[pl.ds(base, GR), :], src_v)
      pltpu.sync_copy(dst_hbm.at[pl.ds(base, GR), :], dst_v)
      def start_gather(r, buf, gsem):
        hh = CHUNK // 2
        pltpu.async_copy(y2.at[src_v.at[r, pl.ds(0, hh)]],
                         buf.at[pl.ds(0, hh), :], gsem)
        pltpu.async_copy(y2.at[src_v.at[r, pl.ds(hh, hh)]],
                         buf.at[pl.ds(hh, hh), :], gsem)

      def wait_gather(r, buf, gsem):
        pltpu.make_async_copy(y2.at[src_v.at[r]], buf, gsem).wait()

      start_gather(0, buf0, gsem0)
      start_gather(1, buf1, gsem1)

      def pair(jj, carry2):
        r0 = jj * 2
        r1 = r0 + 1
        wait_gather(r0, buf0, gsem0)
        pltpu.async_copy(buf0, z_sh.at[dst_v.at[r0]], ssem0, add=True)
        wait_gather(r1, buf1, gsem1)
        pltpu.async_copy(buf1, z_sh.at[dst_v.at[r1]], ssem1, add=True)
        pltpu.make_async_copy(buf0, z_sh.at[dst_v.at[r0]], ssem0).wait()
        start_gather(r0 + 2, buf0, gsem0)
        pltpu.make_async_copy(buf1, z_sh.at[dst_v.at[r1]], ssem1).wait()
        start_gather(r1 + 2, buf1, gsem1)
        return carry2

      lax.fori_loop(0, GR // 2 - 1, pair, carry)
      r0 = GR - 2
      r1 = GR - 1
      wait_gather(r0, buf0, gsem0)
      pltpu.sync_copy(buf0, z_sh.at[dst_v.at[r0]], add=True)
      wait_gather(r1, buf1, gsem1)
      pltpu.sync_copy(buf1, z_sh.at[dst_v.at[r1]], add=True)
      return carry

    lax.fori_loop(0, GROUPS, group, 0)
    plsc.subcore_barrier()
    pltpu.sync_copy(z_sh.at[pl.ds(s * ROWS_T, ROWS_T), :],
                    out_hbm.at[c, pl.ds(s * ROWS_T, ROWS_T), :])

  return k(y, src2d, dst2d, zeros128)


def _dis_from(d0_ref, d1_ref):
  deg = d0_ref[...] + d1_ref[...] + 1.0
  return lax.rsqrt(deg)


def _tc_matmul1(x_pad, W1):
  def body(x_ref, w_ref, xw_ref):
    xw = jnp.dot(x_ref[...], w_ref[...], preferred_element_type=jnp.float32)
    xw_ref[0] = xw[:, :128]
    xw_ref[1] = xw[:, 128:]

  return pl.pallas_call(
      body,
      grid=(NB,),
      in_specs=[
          pl.BlockSpec((BN, D_IN), lambda i: (i, 0)),
          pl.BlockSpec((D_IN, H), lambda i: (0, 0)),
      ],
      out_specs=pl.BlockSpec((NC, BN, 128), lambda i: (0, i, 0)),
      out_shape=jax.ShapeDtypeStruct((NC, N_Z, 128), jnp.float32),
  )(x_pad, W1)


def _tc_scale_y(xw, d0, d1):
  def body(xw_ref, d0_ref, d1_ref, y_ref):
    dis = _dis_from(d0_ref, d1_ref)
    y_ref[0] = dis * xw_ref[0]
    y_ref[1] = dis * xw_ref[1]

  return pl.pallas_call(
      body,
      grid=(NB,),
      in_specs=[
          pl.BlockSpec((NC, BN, 128), lambda i: (0, i, 0)),
          pl.BlockSpec((BN, 1), lambda i: (i, 0)),
          pl.BlockSpec((BN, 1), lambda i: (i, 0)),
      ],
      out_specs=pl.BlockSpec((NC, BN, 128), lambda i: (0, i, 0)),
      out_shape=jax.ShapeDtypeStruct((NC, N_Z, 128), jnp.float32),
  )(xw, d0, d1)


def _tc_boundary(z, xw, d0, d1, b, Wn, relu):
  def body(z_ref, xw_ref, d0_ref, d1_ref, b_ref, w_ref, y_ref, xwn_ref):
    dis = _dis_from(d0_ref, d1_ref)
    z_full = jnp.concatenate([z_ref[0], z_ref[1]], axis=1)
    xw_full = jnp.concatenate([xw_ref[0], xw_ref[1]], axis=1)
    h = dis * z_full + (dis * dis) * xw_full + b_ref[...]
    if relu:
      h = jnp.maximum(h, 0.0)
    xwn = jnp.dot(h, w_ref[...], preferred_element_type=jnp.float32)
    y = dis * xwn
    xwn_ref[0] = xwn[:, :128]
    xwn_ref[1] = xwn[:, 128:]
    y_ref[0] = y[:, :128]
    y_ref[1] = y[:, 128:]

  return pl.pallas_call(
      body,
      grid=(NB,),
      in_specs=[
          pl.BlockSpec((NC, BN, 128), lambda i: (0, i, 0)),
          pl.BlockSpec((NC, BN, 128), lambda i: (0, i, 0)),
          pl.BlockSpec((BN, 1), lambda i: (i, 0)),
          pl.BlockSpec((BN, 1), lambda i: (i, 0)),
          pl.BlockSpec((1, H), lambda i: (0, 0)),
          pl.BlockSpec((H, H), lambda i: (0, 0)),
      ],
      out_specs=[
          pl.BlockSpec((NC, BN, 128), lambda i: (0, i, 0)),
          pl.BlockSpec((NC, BN, 128), lambda i: (0, i, 0)),
      ],
      out_shape=[
          jax.ShapeDtypeStruct((NC, N_Z, 128), jnp.float32),
          jax.ShapeDtypeStruct((NC, N_Z, 128), jnp.float32),
      ],
  )(z, xw, d0, d1, b, Wn)


def _tc_head(z, xw, d0, d1, b3, batch8,
             Wl1, bl1, Wl2a, Wl2b, bl2, Wl3a, Wl3b, bl3):
  def body(z_ref, xw_ref, d0_ref, d1_ref, b3_ref, bt_ref,
           wl1_ref, bl1_ref, wl2a_ref, wl2b_ref, bl2_ref,
           wl3a_ref, wl3b_ref, bl3_ref,
           l1_ref, l2_ref, l3_ref, acc_sum, acc_cnt):
    i = pl.program_id(0)
    dis = _dis_from(d0_ref, d1_ref)
    z_full = jnp.concatenate([z_ref[0], z_ref[1]], axis=1)
    xw_full = jnp.concatenate([xw_ref[0], xw_ref[1]], axis=1)
    h = dis * z_full + (dis * dis) * xw_full + b3_ref[...]

    gids = lax.broadcasted_iota(jnp.int32, (G, 1), 0)
    onehot_t = (bt_ref[0:1, :] == gids).astype(jnp.float32)

    @pl.when(i == 0)
    def _():
      acc_sum[...] = jnp.zeros_like(acc_sum)
      acc_cnt[...] = jnp.zeros_like(acc_cnt)

    acc_sum[...] += jnp.dot(onehot_t, h, preferred_element_type=jnp.float32)
    cnt = jnp.sum(onehot_t, axis=1, keepdims=True)
    acc_cnt[...] += jnp.broadcast_to(cnt, (G, 128))

    @pl.when(i == NB - 1)
    def _():
      pooled = acc_sum[...] / jnp.maximum(acc_cnt[:, 0:1], 1.0)

      def softmax(v):
        m = jnp.max(v, axis=1, keepdims=True)
        e = jnp.exp(v - m)
        return e / jnp.sum(e, axis=1, keepdims=True)

      l1 = jnp.dot(pooled, wl1_ref[...],
                   preferred_element_type=jnp.float32) + bl1_ref[...]
      p1 = softmax(l1)
      l2 = (jnp.dot(pooled, wl2a_ref[...], preferred_element_type=jnp.float32)
            + jnp.dot(p1, wl2b_ref[...], preferred_element_type=jnp.float32)
            + bl2_ref[...])
      p2 = softmax(l2)
      l3 = (jnp.dot(pooled, wl3a_ref[...], preferred_element_type=jnp.float32)
            + jnp.dot(p2, wl3b_ref[...], preferred_element_type=jnp.float32)
            + bl3_ref[...])
      l1_ref[...] = l1
      l2_ref[...] = l2
      l3_ref[...] = l3

  full = lambda shape: pl.BlockSpec(shape, lambda i: tuple(0 for _ in shape))
  return pl.pallas_call(
      body,
      grid=(NB,),
      in_specs=[
          pl.BlockSpec((NC, BN, 128), lambda i: (0, i, 0)),
          pl.BlockSpec((NC, BN, 128), lambda i: (0, i, 0)),
          pl.BlockSpec((BN, 1), lambda i: (i, 0)),
          pl.BlockSpec((BN, 1), lambda i: (i, 0)),
          full((1, H)),
          pl.BlockSpec((8, BN), lambda i: (0, i)),
          full((H, C1)),
          full((1, C1)),
          full((H, C2)),
          full((C1, C2)),
          full((1, C2)),
          full((H, C3)),
          full((C2, C3)),
          full((1, C3)),
      ],
      out_specs=[
          full((G, C1)),
          full((G, C2)),
          full((G, C3)),
      ],
      out_shape=[
          jax.ShapeDtypeStruct((G, C1), jnp.float32),
          jax.ShapeDtypeStruct((G, C2), jnp.float32),
          jax.ShapeDtypeStruct((G, C3), jnp.float32),
      ],
      scratch_shapes=[
          pltpu.VMEM((G, H), jnp.float32),
          pltpu.VMEM((G, 128), jnp.float32),
      ],
  )(z, xw, d0, d1, b3, batch8,
    Wl1, bl1, Wl2a, Wl2b, bl2, Wl3a, Wl3b, bl3)


def kernel(x, edge_index, batch,
           W1, b1, W2, b2, W3, b3, Wl1, bl1, Wl2, bl2, Wl3, bl3):
  src = edge_index[0]
  dst = edge_index[1]
  npad = E_PAD - E
  ar = jnp.arange(npad, dtype=jnp.int32)
  pad_src = (ar * 13) % N
  pad_dst = N + (ar % (N_Z - N))
  src2d = jnp.concatenate([src, pad_src]).reshape(EROWS, CHUNK)
  dst2d = jnp.concatenate([dst, pad_dst]).reshape(EROWS, CHUNK)

  zeros128 = jnp.zeros((N_Z, 128), jnp.float32)
  x_pad = jnp.pad(x, ((0, N_Z - N), (0, 0)))
  batch_pad = jnp.pad(batch, (0, N_Z - N), constant_values=G)
  batch8 = jnp.broadcast_to(batch_pad[None, :], (8, N_Z))

  degs_flat = _sc_degree(dst2d)
  d0 = degs_flat[:N_Z].reshape(N_Z, 1)
  d1 = degs_flat[N_Z:].reshape(N_Z, 1)
  xw = _tc_matmul1(x_pad, W1)
  y = _tc_scale_y(xw, d0, d1)
  z = _sc_edge_sum(y, src2d, dst2d, zeros128)
  y, xw = _tc_boundary(z, xw, d0, d1, b1.reshape(1, H), W2, relu=True)
  z = _sc_edge_sum(y, src2d, dst2d, zeros128)
  y, xw = _tc_boundary(z, xw, d0, d1, b2.reshape(1, H), W3, relu=True)
  z = _sc_edge_sum(y, src2d, dst2d, zeros128)
  l1, l2, l3 = _tc_head(
      z, xw, d0, d1, b3.reshape(1, H), batch8,
      Wl1, bl1.reshape(1, C1), Wl2[:H], Wl2[H:], bl2.reshape(1, C2),
      Wl3[:H], Wl3[H:], bl3.reshape(1, C3))
  return (l1, l2, l3)
